# baseline (speedup 1.0000x reference)
import functools

import jax
import jax.numpy as jnp
from jax import lax
from jax.experimental import pallas as pl
from jax.experimental.pallas import tpu as pltpu
from jax.experimental.pallas import tpu_sc as plsc

VOCAB = 1000000
EMBD = 64
B = 16384
PACK = 128
VROWS = VOCAB // 2

NC = 2
NS = 16
L = 16
NW = NC * NS
BPW = B // NW
CHUNK = 128
NCH = BPW // CHUNK
GPC = CHUNK // L

_mesh = plsc.VectorSubcoreMesh(core_axis_name="c", subcore_axis_name="s")


@functools.partial(
    pl.kernel,
    out_type=jax.ShapeDtypeStruct((NW, BPW), jnp.float32),
    mesh=_mesh,
    compiler_params=pltpu.CompilerParams(needs_layout_passes=False),
    scratch_types=[
        pltpu.VMEM((NCH, CHUNK), jnp.int32),
        pltpu.VMEM((NCH, CHUNK), jnp.int32),
        pltpu.VMEM((NCH, CHUNK), jnp.int32),
        pltpu.VMEM((NCH, CHUNK), jnp.int32),
        pltpu.VMEM((CHUNK, PACK), jnp.float32),
        pltpu.VMEM((CHUNK, PACK), jnp.float32),
        pltpu.VMEM((BPW,), jnp.float32),
        pltpu.SemaphoreType.DMA,
    ],
)
def _skipgram_sc(focus_hbm, context_hbm, emb_hbm, out_hbm,
                 fidx, cidx, fpid, cpid, frows, crows, outv, sem):
    wid = lax.axis_index("s") * NC + lax.axis_index("c")

    pltpu.sync_copy(focus_hbm.at[wid], fidx)
    pltpu.sync_copy(context_hbm.at[wid], cidx)

    for j in range(NCH):
        for k in range(CHUNK // L):
            sl = pl.ds(k * L, L)
            fpid.at[j][sl] = lax.shift_right_logical(fidx.at[j][sl], 1)
            cpid.at[j][sl] = lax.shift_right_logical(cidx.at[j][sl], 1)

    iota = lax.iota(jnp.int32, L)

    for j in range(NCH):
        fcp = pltpu.async_copy(emb_hbm.at[fpid.at[j]], frows, sem)
        ccp = pltpu.async_copy(emb_hbm.at[cpid.at[j]], crows, sem)
        fcp.wait()
        ccp.wait()

        def body(g, _, j=j):
            row = g * L + iota
            sl = pl.ds(g * L, L)
            colf = (fidx.at[j][sl] & 1) * EMBD
            colc = (cidx.at[j][sl] & 1) * EMBD
            acc = jnp.zeros((L,), jnp.float32)
            for d in range(EMBD):
                f = plsc.load_gather(frows, [row, colf])
                c = plsc.load_gather(crows, [row, colc])
                acc = acc + f * c
                if d != EMBD - 1:
                    colf = colf + 1
                    colc = colc + 1
            outv[pl.ds(j * CHUNK + g * L, L)] = acc
            return _

        lax.fori_loop(0, GPC, body, None)

    pltpu.sync_copy(outv, out_hbm.at[wid])


def kernel(focus, context, embeddings):
    emb2 = embeddings.reshape(VROWS, PACK)
    focus = focus.reshape(NW, NCH, CHUNK)
    context = context.reshape(NW, NCH, CHUNK)
    out = _skipgram_sc(focus, context, emb2)
    return out.reshape(B)
